# Initial kernel scaffold; baseline (speedup 1.0000x reference)
import jax
import jax.numpy as jnp
from jax.experimental import pallas as pl


def kernel(additional_x, edge_index, batch, W_root0, W_neigh0, b0, W_root1, W_neigh1, b1, W_root2, W_neigh2, b2, Wi1, bi1, Wi2, bi2, Wi3, bi3, Wo, bo):
    raise NotImplementedError("write your pallas kernel here")


# speedup vs baseline: 3.6313x; 3.6313x over previous
import functools

import jax
import jax.numpy as jnp
from jax import lax
from jax.experimental import pallas as pl
from jax.experimental.pallas import tpu as pltpu
from jax.experimental.pallas import tpu_sc as plsc

N = 10000
E = 320000
NHID = 128
EMBED = 64
NGRAPHS = 64

NP = 10240
ER = 2560
EP = ER * 128
DUMMY = N
NT = 16
RPW = ER // 32
RT = NP // NT

_HIGH = jax.lax.Precision.HIGHEST


def _dot(a, b):
    return jax.lax.dot_general(a, b, (((1,), (0,)), ((), ())),
                               precision=_HIGH,
                               preferred_element_type=jnp.float32)


@functools.lru_cache(maxsize=None)
def _make_sc_spmm(D):
    mesh = plsc.VectorSubcoreMesh(core_axis_name="c", subcore_axis_name="s",
                                  num_cores=2, num_subcores=NT)

    scratch = [
        pltpu.VMEM((RPW, 128), jnp.int32),
        pltpu.VMEM((2, 128), jnp.int32),
        pltpu.VMEM((128, D), jnp.float32),
        pltpu.VMEM_SHARED((NP, D), jnp.float32),
        pltpu.SemaphoreType.DMA,
    ]

    @functools.partial(
        pl.kernel,
        out_type=jax.ShapeDtypeStruct((2, NP, D), jnp.float32),
        mesh=mesh,
        scratch_types=scratch,
    )
    def spmm(h_hbm, edges_hbm, out_hbm, ipk, idx, buf, acc, sem):
        c = lax.axis_index("c")
        s = lax.axis_index("s")
        w = c * NT + s
        zbase = s * RT
        tbl = h_hbm

        ebase = w * RPW
        pltpu.sync_copy(edges_hbm.at[pl.ds(ebase, RPW)], ipk)

        def _zrow(i, carry):
            for kk in range(D // 16):
                buf[i, pl.ds(kk * 16, 16)] = jnp.zeros((16,), jnp.float32)
            return carry
        lax.fori_loop(0, 128, _zrow, 0)
        for i in range(RT // 128):
            pltpu.sync_copy(buf, acc.at[pl.ds(zbase + i * 128, 128)])
        plsc.subcore_barrier()

        def _step(j, carry):
            for kk in range(128 // 16):
                p = ipk[j, pl.ds(kk * 16, 16)]
                idx[0, pl.ds(kk * 16, 16)] = p >> 14
                idx[1, pl.ds(kk * 16, 16)] = p & 16383
            pltpu.async_copy(tbl.at[idx.at[0]], buf, sem).wait()
            pltpu.sync_copy(buf, acc.at[idx.at[1]], add=True)
            return carry
        lax.fori_loop(0, RPW, _step, 0)
        plsc.subcore_barrier()

        for i in range(RT // 128):
            r0 = zbase + i * 128
            pltpu.sync_copy(acc.at[pl.ds(r0, 128)], buf)
            pltpu.sync_copy(buf, out_hbm.at[c].at[pl.ds(r0, 128)])

    return spmm


def _sc_spmm128(h, edges):
    return _make_sc_spmm(128)(h, edges)


def _l0_body(x_ref, p_ref, wr_ref, wn_ref, b_ref, o_ref):
    a = p_ref[0, :, 0:1] + p_ref[1, :, 0:1]
    x = x_ref[:, 0:1]
    o_ref[...] = jnp.maximum(
        x * wr_ref[...] + a * wn_ref[...] + b_ref[...], 0.0)


def _tc_l0(x128, p0, wr, wn, b):
    B = 256
    return pl.pallas_call(
        _l0_body,
        grid=(NP // B,),
        in_specs=[
            pl.BlockSpec((B, NHID), lambda i: (i, 0)),
            pl.BlockSpec((2, B, NHID), lambda i: (0, i, 0)),
            pl.BlockSpec((1, NHID), lambda i: (0, 0)),
            pl.BlockSpec((1, NHID), lambda i: (0, 0)),
            pl.BlockSpec((1, NHID), lambda i: (0, 0)),
        ],
        out_specs=pl.BlockSpec((B, NHID), lambda i: (i, 0)),
        out_shape=jax.ShapeDtypeStruct((NP, NHID), jnp.float32),
    )(x128, p0, wr.reshape(1, NHID), wn.reshape(1, NHID), b.reshape(1, NHID))


def _conv_body(h_ref, p_ref, wr_ref, wn_ref, b_ref, o_ref):
    agg = p_ref[0] + p_ref[1]
    o_ref[...] = jnp.maximum(
        _dot(h_ref[...], wr_ref[...]) + _dot(agg, wn_ref[...]) + b_ref[...],
        0.0)


def _tc_conv(h, p, wr, wn, b):
    B = 256
    return pl.pallas_call(
        _conv_body,
        grid=(NP // B,),
        in_specs=[
            pl.BlockSpec((B, NHID), lambda i: (i, 0)),
            pl.BlockSpec((2, B, NHID), lambda i: (0, i, 0)),
            pl.BlockSpec((NHID, NHID), lambda i: (0, 0)),
            pl.BlockSpec((NHID, NHID), lambda i: (0, 0)),
            pl.BlockSpec((1, NHID), lambda i: (0, 0)),
        ],
        out_specs=pl.BlockSpec((B, NHID), lambda i: (i, 0)),
        out_shape=jax.ShapeDtypeStruct((NP, NHID), jnp.float32),
    )(h, p, wr, wn, b.reshape(1, NHID))


def _mlp_body(h1_ref, h2_ref, h3_ref, bt_ref, wi1_ref, bi1_ref, wi2_ref,
              bi2_ref, wi3_ref, bi3_ref, wot_ref, bo_ref, o_ref,
              pool_ref, cnt_ref, mm_ref):
    i = pl.program_id(0)
    B = h1_ref.shape[0]
    wi1 = wi1_ref[...]
    z = jnp.maximum(
        _dot(h1_ref[...], wi1[0:NHID])
        + _dot(h2_ref[...], wi1[NHID:2 * NHID])
        + _dot(h3_ref[...], wi1[2 * NHID:3 * NHID]) + bi1_ref[...], 0.0)
    z = jnp.maximum(_dot(z, wi2_ref[...]) + bi2_ref[...], 0.0)
    z = _dot(z, wi3_ref[...]) + bi3_ref[...]

    rows = i * B + jax.lax.broadcasted_iota(jnp.int32, (B, 1), 0)
    valid = rows < N
    big = jnp.float32(3.4e38)
    mn_blk = jnp.min(jnp.where(valid, z, big))
    mx_blk = jnp.max(jnp.where(valid, z, -big))

    g_iota = jax.lax.broadcasted_iota(jnp.int32, (B, NGRAPHS), 1)
    oh = jnp.where((bt_ref[...] == g_iota) & valid, 1.0, 0.0)
    pool_blk = jax.lax.dot_general(oh, z, (((0,), (0,)), ((), ())),
                                   precision=_HIGH,
                                   preferred_element_type=jnp.float32)
    cnt_blk = jnp.sum(oh, axis=0).reshape(NGRAPHS, 1)

    @pl.when(i == 0)
    def _init():
        pool_ref[...] = pool_blk
        cnt_ref[...] = cnt_blk
        mm_ref[0] = mn_blk
        mm_ref[1] = mx_blk

    @pl.when(i > 0)
    def _acc():
        pool_ref[...] += pool_blk
        cnt_ref[...] += cnt_blk
        mm_ref[0] = jnp.minimum(mm_ref[0], mn_blk)
        mm_ref[1] = jnp.maximum(mm_ref[1], mx_blk)

    @pl.when(i == pl.num_programs(0) - 1)
    def _fin():
        mn = mm_ref[0]
        mx = mm_ref[1]
        scale = 2.0 / (mx - mn)
        pooln = (pool_ref[...] - mn * cnt_ref[...]) * scale
        o_ref[...] = (jnp.sum(pooln * wot_ref[...], axis=1, keepdims=True)
                      + bo_ref[...])


def _tc_mlp(h1, h2, h3, batch2d, wi1, bi1, wi2, bi2, wi3, bi3, wo, bo):
    B = 256
    return pl.pallas_call(
        _mlp_body,
        grid=(NP // B,),
        in_specs=[
            pl.BlockSpec((B, NHID), lambda i: (i, 0)),
            pl.BlockSpec((B, NHID), lambda i: (i, 0)),
            pl.BlockSpec((B, NHID), lambda i: (i, 0)),
            pl.BlockSpec((B, 1), lambda i: (i, 0)),
            pl.BlockSpec((3 * NHID, NHID), lambda i: (0, 0)),
            pl.BlockSpec((1, NHID), lambda i: (0, 0)),
            pl.BlockSpec((NHID, NHID), lambda i: (0, 0)),
            pl.BlockSpec((1, NHID), lambda i: (0, 0)),
            pl.BlockSpec((NHID, EMBED), lambda i: (0, 0)),
            pl.BlockSpec((1, EMBED), lambda i: (0, 0)),
            pl.BlockSpec((1, EMBED), lambda i: (0, 0)),
            pl.BlockSpec((1, 1), lambda i: (0, 0)),
        ],
        out_specs=pl.BlockSpec((NGRAPHS, 1), lambda i: (0, 0)),
        out_shape=jax.ShapeDtypeStruct((NGRAPHS, 1), jnp.float32),
        scratch_shapes=[
            pltpu.VMEM((NGRAPHS, EMBED), jnp.float32),
            pltpu.VMEM((NGRAPHS, 1), jnp.float32),
            pltpu.SMEM((2,), jnp.float32),
        ],
    )(h1, h2, h3, batch2d, wi1, bi1.reshape(1, NHID), wi2,
      bi2.reshape(1, NHID), wi3, bi3.reshape(1, EMBED), wo.reshape(1, EMBED),
      bo.reshape(1, 1))


def kernel(additional_x, edge_index, batch,
           W_root0, W_neigh0, b0,
           W_root1, W_neigh1, b1,
           W_root2, W_neigh2, b2,
           Wi1, bi1, Wi2, bi2, Wi3, bi3, Wo, bo):
    x = additional_x.reshape(N, 1)
    src = edge_index[0]
    dst = edge_index[1]
    pad = EP - E
    packed = (src << 14) | dst
    edges = jnp.concatenate(
        [packed, jnp.full((pad,), DUMMY, jnp.int32)]).reshape(ER, 128)
    x128 = jnp.zeros((NP, NHID), jnp.float32).at[:N, 0].set(x[:, 0])
    batch2d = jnp.full((NP, 1), NGRAPHS, jnp.int32).at[:N, 0].set(batch)

    p0 = _sc_spmm128(x128, edges)
    h1 = _tc_l0(x128, p0, W_root0, W_neigh0, b0)
    p1 = _sc_spmm128(h1, edges)
    h2 = _tc_conv(h1, p1, W_root1, W_neigh1, b1)
    p2 = _sc_spmm128(h2, edges)
    h3 = _tc_conv(h2, p2, W_root2, W_neigh2, b2)
    return _tc_mlp(h1, h2, h3, batch2d, Wi1, bi1, Wi2, bi2, Wi3, bi3, Wo, bo)


# speedup vs baseline: 4.1183x; 1.1341x over previous
import functools

import jax
import jax.numpy as jnp
from jax import lax
from jax.experimental import pallas as pl
from jax.experimental.pallas import tpu as pltpu
from jax.experimental.pallas import tpu_sc as plsc

N = 10000
E = 320000
NHID = 128
EMBED = 64
NGRAPHS = 64

NP = 10240
ER = 2560
EP = ER * 128
DUMMY = N
NT = 16
RPW = ER // 32
RT = NP // NT

_HIGH = jax.lax.Precision.HIGHEST


def _dot(a, b):
    return jax.lax.dot_general(a, b, (((1,), (0,)), ((), ())),
                               precision=_HIGH,
                               preferred_element_type=jnp.float32)


@functools.lru_cache(maxsize=None)
def _make_sc_spmm(D):
    mesh = plsc.VectorSubcoreMesh(core_axis_name="c", subcore_axis_name="s",
                                  num_cores=2, num_subcores=NT)

    scratch = [
        pltpu.VMEM((RPW, 128), jnp.int32),
        pltpu.VMEM((4, 128), jnp.int32),
        pltpu.VMEM((128, D), jnp.float32),
        pltpu.VMEM((128, D), jnp.float32),
        pltpu.VMEM_SHARED((NP, D), jnp.float32),
        pltpu.SemaphoreType.DMA,
        pltpu.SemaphoreType.DMA,
    ]

    @functools.partial(
        pl.kernel,
        out_type=jax.ShapeDtypeStruct((2, NP, D), jnp.float32),
        mesh=mesh,
        scratch_types=scratch,
    )
    def spmm(h_hbm, edges_hbm, out_hbm, ipk, idx, buf0, buf1, acc,
             sem0, sem1):
        c = lax.axis_index("c")
        s = lax.axis_index("s")
        w = c * NT + s
        zbase = s * RT
        tbl = h_hbm

        ebase = w * RPW
        pltpu.sync_copy(edges_hbm.at[pl.ds(ebase, RPW)], ipk)

        def _unpack(j, slot):
            for kk in range(128 // 16):
                p = ipk[j, pl.ds(kk * 16, 16)]
                idx[2 * slot, pl.ds(kk * 16, 16)] = p >> 14
                idx[2 * slot + 1, pl.ds(kk * 16, 16)] = p & 16383

        def _zrow(i, carry):
            for kk in range(D // 16):
                buf0[i, pl.ds(kk * 16, 16)] = jnp.zeros((16,), jnp.float32)
            return carry
        lax.fori_loop(0, 128, _zrow, 0)
        for i in range(RT // 128):
            pltpu.sync_copy(buf0, acc.at[pl.ds(zbase + i * 128, 128)])
        plsc.subcore_barrier()

        _unpack(0, 0)
        pltpu.async_copy(tbl.at[idx.at[0]], buf0, sem0)

        def _pair(i, carry):
            _unpack(2 * i + 1, 1)
            pltpu.async_copy(tbl.at[idx.at[2]], buf1, sem1)
            pltpu.make_async_copy(tbl.at[idx.at[0]], buf0, sem0).wait()
            pltpu.sync_copy(buf0, acc.at[idx.at[1]], add=True)

            @pl.when(i < RPW // 2 - 1)
            def _next():
                _unpack(2 * i + 2, 0)
                pltpu.async_copy(tbl.at[idx.at[0]], buf0, sem0)

            pltpu.make_async_copy(tbl.at[idx.at[2]], buf1, sem1).wait()
            pltpu.sync_copy(buf1, acc.at[idx.at[3]], add=True)
            return carry
        lax.fori_loop(0, RPW // 2, _pair, 0)
        plsc.subcore_barrier()

        for i in range(RT // 128):
            r0 = zbase + i * 128
            pltpu.sync_copy(acc.at[pl.ds(r0, 128)], buf0)
            pltpu.sync_copy(buf0, out_hbm.at[c].at[pl.ds(r0, 128)])

    return spmm


def _sc_spmm128(h, edges):
    return _make_sc_spmm(128)(h, edges)


def _l0_body(x_ref, p_ref, wr_ref, wn_ref, b_ref, o_ref):
    a = p_ref[0, :, 0:1] + p_ref[1, :, 0:1]
    x = x_ref[:, 0:1]
    o_ref[...] = jnp.maximum(
        x * wr_ref[...] + a * wn_ref[...] + b_ref[...], 0.0)


def _tc_l0(x128, p0, wr, wn, b):
    B = 256
    return pl.pallas_call(
        _l0_body,
        grid=(NP // B,),
        in_specs=[
            pl.BlockSpec((B, NHID), lambda i: (i, 0)),
            pl.BlockSpec((2, B, NHID), lambda i: (0, i, 0)),
            pl.BlockSpec((1, NHID), lambda i: (0, 0)),
            pl.BlockSpec((1, NHID), lambda i: (0, 0)),
            pl.BlockSpec((1, NHID), lambda i: (0, 0)),
        ],
        out_specs=pl.BlockSpec((B, NHID), lambda i: (i, 0)),
        out_shape=jax.ShapeDtypeStruct((NP, NHID), jnp.float32),
    )(x128, p0, wr.reshape(1, NHID), wn.reshape(1, NHID), b.reshape(1, NHID))


def _conv_body(h_ref, p_ref, wr_ref, wn_ref, b_ref, o_ref):
    agg = p_ref[0] + p_ref[1]
    o_ref[...] = jnp.maximum(
        _dot(h_ref[...], wr_ref[...]) + _dot(agg, wn_ref[...]) + b_ref[...],
        0.0)


def _tc_conv(h, p, wr, wn, b):
    B = 256
    return pl.pallas_call(
        _conv_body,
        grid=(NP // B,),
        in_specs=[
            pl.BlockSpec((B, NHID), lambda i: (i, 0)),
            pl.BlockSpec((2, B, NHID), lambda i: (0, i, 0)),
            pl.BlockSpec((NHID, NHID), lambda i: (0, 0)),
            pl.BlockSpec((NHID, NHID), lambda i: (0, 0)),
            pl.BlockSpec((1, NHID), lambda i: (0, 0)),
        ],
        out_specs=pl.BlockSpec((B, NHID), lambda i: (i, 0)),
        out_shape=jax.ShapeDtypeStruct((NP, NHID), jnp.float32),
    )(h, p, wr, wn, b.reshape(1, NHID))


def _mlp_body(h1_ref, h2_ref, h3_ref, bt_ref, wi1_ref, bi1_ref, wi2_ref,
              bi2_ref, wi3_ref, bi3_ref, wot_ref, bo_ref, o_ref,
              pool_ref, cnt_ref, mm_ref):
    i = pl.program_id(0)
    B = h1_ref.shape[0]
    wi1 = wi1_ref[...]
    z = jnp.maximum(
        _dot(h1_ref[...], wi1[0:NHID])
        + _dot(h2_ref[...], wi1[NHID:2 * NHID])
        + _dot(h3_ref[...], wi1[2 * NHID:3 * NHID]) + bi1_ref[...], 0.0)
    z = jnp.maximum(_dot(z, wi2_ref[...]) + bi2_ref[...], 0.0)
    z = _dot(z, wi3_ref[...]) + bi3_ref[...]

    rows = i * B + jax.lax.broadcasted_iota(jnp.int32, (B, 1), 0)
    valid = rows < N
    big = jnp.float32(3.4e38)
    mn_blk = jnp.min(jnp.where(valid, z, big))
    mx_blk = jnp.max(jnp.where(valid, z, -big))

    g_iota = jax.lax.broadcasted_iota(jnp.int32, (B, NGRAPHS), 1)
    oh = jnp.where((bt_ref[...] == g_iota) & valid, 1.0, 0.0)
    pool_blk = jax.lax.dot_general(oh, z, (((0,), (0,)), ((), ())),
                                   precision=_HIGH,
                                   preferred_element_type=jnp.float32)
    cnt_blk = jnp.sum(oh, axis=0).reshape(NGRAPHS, 1)

    @pl.when(i == 0)
    def _init():
        pool_ref[...] = pool_blk
        cnt_ref[...] = cnt_blk
        mm_ref[0] = mn_blk
        mm_ref[1] = mx_blk

    @pl.when(i > 0)
    def _acc():
        pool_ref[...] += pool_blk
        cnt_ref[...] += cnt_blk
        mm_ref[0] = jnp.minimum(mm_ref[0], mn_blk)
        mm_ref[1] = jnp.maximum(mm_ref[1], mx_blk)

    @pl.when(i == pl.num_programs(0) - 1)
    def _fin():
        mn = mm_ref[0]
        mx = mm_ref[1]
        scale = 2.0 / (mx - mn)
        pooln = (pool_ref[...] - mn * cnt_ref[...]) * scale
        o_ref[...] = (jnp.sum(pooln * wot_ref[...], axis=1, keepdims=True)
                      + bo_ref[...])


def _tc_mlp(h1, h2, h3, batch2d, wi1, bi1, wi2, bi2, wi3, bi3, wo, bo):
    B = 256
    return pl.pallas_call(
        _mlp_body,
        grid=(NP // B,),
        in_specs=[
            pl.BlockSpec((B, NHID), lambda i: (i, 0)),
            pl.BlockSpec((B, NHID), lambda i: (i, 0)),
            pl.BlockSpec((B, NHID), lambda i: (i, 0)),
            pl.BlockSpec((B, 1), lambda i: (i, 0)),
            pl.BlockSpec((3 * NHID, NHID), lambda i: (0, 0)),
            pl.BlockSpec((1, NHID), lambda i: (0, 0)),
            pl.BlockSpec((NHID, NHID), lambda i: (0, 0)),
            pl.BlockSpec((1, NHID), lambda i: (0, 0)),
            pl.BlockSpec((NHID, EMBED), lambda i: (0, 0)),
            pl.BlockSpec((1, EMBED), lambda i: (0, 0)),
            pl.BlockSpec((1, EMBED), lambda i: (0, 0)),
            pl.BlockSpec((1, 1), lambda i: (0, 0)),
        ],
        out_specs=pl.BlockSpec((NGRAPHS, 1), lambda i: (0, 0)),
        out_shape=jax.ShapeDtypeStruct((NGRAPHS, 1), jnp.float32),
        scratch_shapes=[
            pltpu.VMEM((NGRAPHS, EMBED), jnp.float32),
            pltpu.VMEM((NGRAPHS, 1), jnp.float32),
            pltpu.SMEM((2,), jnp.float32),
        ],
    )(h1, h2, h3, batch2d, wi1, bi1.reshape(1, NHID), wi2,
      bi2.reshape(1, NHID), wi3, bi3.reshape(1, EMBED), wo.reshape(1, EMBED),
      bo.reshape(1, 1))


def kernel(additional_x, edge_index, batch,
           W_root0, W_neigh0, b0,
           W_root1, W_neigh1, b1,
           W_root2, W_neigh2, b2,
           Wi1, bi1, Wi2, bi2, Wi3, bi3, Wo, bo):
    x = additional_x.reshape(N, 1)
    src = edge_index[0]
    dst = edge_index[1]
    pad = EP - E
    packed = (src << 14) | dst
    edges = jnp.concatenate(
        [packed, jnp.full((pad,), DUMMY, jnp.int32)]).reshape(ER, 128)
    x128 = jnp.zeros((NP, NHID), jnp.float32).at[:N, 0].set(x[:, 0])
    batch2d = jnp.full((NP, 1), NGRAPHS, jnp.int32).at[:N, 0].set(batch)

    p0 = _sc_spmm128(x128, edges)
    h1 = _tc_l0(x128, p0, W_root0, W_neigh0, b0)
    p1 = _sc_spmm128(h1, edges)
    h2 = _tc_conv(h1, p1, W_root1, W_neigh1, b1)
    p2 = _sc_spmm128(h2, edges)
    h3 = _tc_conv(h2, p2, W_root2, W_neigh2, b2)
    return _tc_mlp(h1, h2, h3, batch2d, Wi1, bi1, Wi2, bi2, Wi3, bi3, Wo, bo)
